# baseline (speedup 1.0000x reference)
import dataclasses
import functools

import jax
import jax.numpy as jnp
from jax import lax
from jax.experimental import pallas as pl
from jax.experimental.pallas import tpu as pltpu
from jax.experimental.pallas import tpu_sc as plsc

BATCH = 16384
FIELDS = 100
EDIM = 16
EMBED = FIELDS * EDIM
K = 160
NC = 2
NS = 16
L = 16
NW = NC * NS
LANES_PER_TILE = BATCH // NW
G = 128
NGRP = LANES_PER_TILE // G
NB = 4
BAND = EMBED // NB
NBUF = 2
CHUNKS = G // L
NSLAB = NGRP * NB


def _sc_mask_copy(ev, iv):
  mesh = plsc.VectorSubcoreMesh(core_axis_name="c", subcore_axis_name="s")
  cp = pltpu.CompilerParams()
  if "needs_layout_passes" in pltpu.CompilerParams.__dataclass_fields__:
    cp = dataclasses.replace(cp, needs_layout_passes=False)

  @functools.partial(
      pl.kernel,
      compiler_params=cp,
      out_type=jax.ShapeDtypeStruct((EMBED, BATCH), jnp.float32),
      mesh=mesh,
      scratch_types=(
          [pltpu.VMEM((BAND + 1, G), jnp.float32) for _ in range(NBUF)]
          + [pltpu.VMEM((K, G), jnp.int32)]
          + [pltpu.SemaphoreType.DMA] * (2 * NBUF + 1)
      ),
  )
  def k(ev_hbm, iv_hbm, out_hbm, d0, d1, ibuf, sd0, sd1, so0, so1, si):
    dbufs = (d0, d1)
    sds, sos = (sd0, sd1), (so0, so1)
    wid = lax.axis_index("s") * NC + lax.axis_index("c")
    lane0 = wid * LANES_PER_TILE

    def slab(s):
      grp, band = s // NB, s % NB
      return lane0 + grp * G, band * BAND

    def in_copy(s, b):
      base, p0 = slab(s)
      return pltpu.make_async_copy(
          ev_hbm.at[pl.ds(p0, BAND), pl.ds(base, G)],
          dbufs[b].at[pl.ds(0, BAND)], sds[b])

    def out_copy(s, b):
      base, p0 = slab(s)
      return pltpu.make_async_copy(
          dbufs[b].at[pl.ds(0, BAND)],
          out_hbm.at[pl.ds(p0, BAND), pl.ds(base, G)], sos[b])

    def idx_copy(grp):
      return pltpu.make_async_copy(
          iv_hbm.at[:, pl.ds(lane0 + grp * G, G)], ibuf, si)

    def scatter_zeros_pair(band0):
      zeros = jnp.zeros((L,), jnp.float32)
      p0s = [jnp.uint32((band0 + i) * BAND) for i in range(NBUF)]
      cap = jnp.uint32(BAND)
      lanes = [lax.iota(jnp.int32, L) + c * L for c in range(CHUNKS)]

      @pl.loop(0, K)
      def _(r):
        rows = [ibuf[r, pl.ds(c * L, L)] for c in range(CHUNKS)]
        us = [lax.bitcast_convert_type(rv, jnp.uint32) for rv in rows]
        for c in range(CHUNKS):
          for i in range(NBUF):
            local = lax.bitcast_convert_type(
                jnp.minimum(us[c] - p0s[i], cap), jnp.int32)
            plsc.store_scatter(dbufs[i], [local, lanes[c]], zeros)

    idx_copy(0).start()
    in_copy(0, 0).start()
    in_copy(1, 1).start()
    for s0 in range(0, NSLAB, NBUF):
      for b in range(NBUF):
        in_copy(s0 + b, b).wait()
      if s0 % NB == 0:
        idx_copy(s0 // NB).wait()
      scatter_zeros_pair(s0 % NB)
      if (s0 + 1) % NB == NB - 1 and (s0 + 1) // NB + 1 < NGRP:
        idx_copy((s0 + 1) // NB + 1).start()
      for b in range(NBUF):
        out_copy(s0 + b, b).start()
      for b in range(NBUF):
        s = s0 + b
        out_copy(s, b).wait()
        if s + NBUF < NSLAB:
          in_copy(s + NBUF, b).start()

  return k(ev, iv)


def kernel(embed, indexes, mask_num):
  del mask_num
  ev = jnp.transpose(embed, (1, 2, 0)).reshape(EMBED, BATCH)
  iv = jnp.transpose(indexes.astype(jnp.int32), (1, 0))
  ov = _sc_mask_copy(ev, iv)
  return jnp.transpose(ov.reshape(FIELDS, EDIM, BATCH), (2, 0, 1))


# speedup vs baseline: 1.1673x; 1.1673x over previous
import dataclasses
import functools

import jax
import jax.numpy as jnp
from jax import lax
from jax.experimental import pallas as pl
from jax.experimental.pallas import tpu as pltpu
from jax.experimental.pallas import tpu_sc as plsc

BATCH = 16384
FIELDS = 100
EDIM = 16
EMBED = FIELDS * EDIM
K = 160
NC = 2
NS = 16
L = 16
NW = NC * NS
LANES_PER_TILE = BATCH // NW
G = 128
NGRP = LANES_PER_TILE // G
NB = 4
BAND = EMBED // NB
NBUF = 2
CHUNKS = G // L
NSLAB = NGRP * NB


def _sc_mask_copy(ev, iv):
  mesh = plsc.VectorSubcoreMesh(core_axis_name="c", subcore_axis_name="s")
  cp = pltpu.CompilerParams()
  if "needs_layout_passes" in pltpu.CompilerParams.__dataclass_fields__:
    cp = dataclasses.replace(cp, needs_layout_passes=False)

  @functools.partial(
      pl.kernel,
      compiler_params=cp,
      out_type=jax.ShapeDtypeStruct((EMBED, BATCH), jnp.float32),
      mesh=mesh,
      scratch_types=(
          [pltpu.VMEM((BAND + 1, G), jnp.float32) for _ in range(NBUF)]
          + [pltpu.VMEM((K, G), jnp.int32)]
          + [pltpu.SemaphoreType.DMA] * (2 * NBUF + 1)
      ),
  )
  def k(ev_hbm, iv_hbm, out_hbm, d0, d1, ibuf, sd0, sd1, so0, so1, si):
    dbufs = (d0, d1)
    sds, sos = (sd0, sd1), (so0, so1)
    wid = lax.axis_index("s") * NC + lax.axis_index("c")
    lane0 = wid * LANES_PER_TILE

    def slab(s):
      grp, band = s // NB, s % NB
      return lane0 + grp * G, band * BAND

    def in_copy(s, b):
      base, p0 = slab(s)
      return pltpu.make_async_copy(
          ev_hbm.at[pl.ds(p0, BAND), pl.ds(base, G)],
          dbufs[b].at[pl.ds(0, BAND)], sds[b])

    def out_copy(s, b):
      base, p0 = slab(s)
      return pltpu.make_async_copy(
          dbufs[b].at[pl.ds(0, BAND)],
          out_hbm.at[pl.ds(p0, BAND), pl.ds(base, G)], sos[b])

    def idx_copy(grp):
      return pltpu.make_async_copy(
          iv_hbm.at[:, pl.ds(lane0 + grp * G, G)], ibuf, si)

    def scatter_zeros(b, band):
      zeros = jnp.zeros((L,), jnp.float32)
      p0 = jnp.uint32(band * BAND)
      cap = jnp.uint32(BAND)
      lanes = [lax.iota(jnp.int32, L) + c * L for c in range(CHUNKS)]

      @pl.loop(0, K)
      def _(r):
        rows = [ibuf[r, pl.ds(c * L, L)] for c in range(CHUNKS)]
        us = [lax.bitcast_convert_type(rv, jnp.uint32) - p0 for rv in rows]
        for c in range(CHUNKS):
          local = lax.bitcast_convert_type(jnp.minimum(us[c], cap), jnp.int32)
          plsc.store_scatter(dbufs[b], [local, lanes[c]], zeros)

    idx_copy(0).start()
    in_copy(0, 0).start()
    in_copy(1, 1).start()
    for s0 in range(0, NSLAB, NBUF):
      for b in range(NBUF):
        s = s0 + b
        in_copy(s, b).wait()
        if s % NB == 0:
          idx_copy(s // NB).wait()
        scatter_zeros(b, s % NB)
        if s % NB == NB - 1 and s // NB + 1 < NGRP:
          idx_copy(s // NB + 1).start()
        out_copy(s, b).start()
      for b in range(NBUF):
        s = s0 + b
        out_copy(s, b).wait()
        if s + NBUF < NSLAB:
          in_copy(s + NBUF, b).start()

  return k(ev, iv)


def kernel(embed, indexes, mask_num):
  del mask_num
  ev = jnp.transpose(embed, (1, 2, 0)).reshape(EMBED, BATCH)
  iv = jnp.transpose(indexes.astype(jnp.int32), (1, 0))
  ov = _sc_mask_copy(ev, iv)
  return jnp.transpose(ov.reshape(FIELDS, EDIM, BATCH), (2, 0, 1))


# speedup vs baseline: 1.1780x; 1.0092x over previous
import dataclasses
import functools

import jax
import jax.numpy as jnp
from jax import lax
from jax.experimental import pallas as pl
from jax.experimental.pallas import tpu as pltpu
from jax.experimental.pallas import tpu_sc as plsc

BATCH = 16384
FIELDS = 100
EDIM = 16
EMBED = FIELDS * EDIM
K = 160
NC = 2
NS = 16
L = 16
NW = NC * NS
LANES_PER_TILE = BATCH // NW
G = 128
NGRP = LANES_PER_TILE // G
NB = 4
BAND = EMBED // NB
NBUF = 2
CHUNKS = G // L
NSLAB = NGRP * NB


def _sc_mask_copy(ev, iv):
  mesh = plsc.VectorSubcoreMesh(core_axis_name="c", subcore_axis_name="s")
  cp = pltpu.CompilerParams()
  if "needs_layout_passes" in pltpu.CompilerParams.__dataclass_fields__:
    cp = dataclasses.replace(cp, needs_layout_passes=False)

  @functools.partial(
      pl.kernel,
      compiler_params=cp,
      out_type=jax.ShapeDtypeStruct((EMBED, BATCH), jnp.float32),
      mesh=mesh,
      scratch_types=(
          [pltpu.VMEM((BAND + 1, G), jnp.float32) for _ in range(NBUF)]
          + [pltpu.VMEM((K, G), jnp.int32)]
          + [pltpu.SemaphoreType.DMA] * (2 * NBUF + 1)
      ),
  )
  def k(ev_hbm, iv_hbm, out_hbm, d0, d1, ibuf, sd0, sd1, so0, so1, si):
    dbufs = (d0, d1)
    sds, sos = (sd0, sd1), (so0, so1)
    wid = lax.axis_index("s") * NC + lax.axis_index("c")
    lane0 = wid * LANES_PER_TILE

    def slab(s):
      grp, band = s // NB, s % NB
      return lane0 + grp * G, band * BAND

    def in_copy(s, b):
      base, p0 = slab(s)
      return pltpu.make_async_copy(
          ev_hbm.at[pl.ds(p0, BAND), pl.ds(base, G)],
          dbufs[b].at[pl.ds(0, BAND)], sds[b])

    def out_copy(s, b):
      base, p0 = slab(s)
      return pltpu.make_async_copy(
          dbufs[b].at[pl.ds(0, BAND)],
          out_hbm.at[pl.ds(p0, BAND), pl.ds(base, G)], sos[b])

    def idx_copy(grp):
      return pltpu.make_async_copy(
          iv_hbm.at[:, pl.ds(lane0 + grp * G, G)], ibuf, si)

    def scatter_zeros(b, band):
      zeros = jnp.zeros((L,), jnp.float32)
      p0 = jnp.uint32(band * BAND)
      cap = jnp.uint32(BAND)
      lanes = [lax.iota(jnp.int32, L) + c * L for c in range(CHUNKS)]

      @pl.loop(0, K)
      def _(r):
        rows = [ibuf[r, pl.ds(c * L, L)] for c in range(CHUNKS)]
        us = [lax.bitcast_convert_type(rv, jnp.uint32) - p0 for rv in rows]
        for c in range(CHUNKS):
          local = lax.bitcast_convert_type(us[c], jnp.int32)
          plsc.store_scatter(dbufs[b], [local, lanes[c]], zeros,
                             mask=us[c] < cap)

    idx_copy(0).start()
    in_copy(0, 0).start()
    in_copy(1, 1).start()
    for s0 in range(0, NSLAB, NBUF):
      for b in range(NBUF):
        s = s0 + b
        in_copy(s, b).wait()
        if s % NB == 0:
          idx_copy(s // NB).wait()
        scatter_zeros(b, s % NB)
        if s % NB == NB - 1 and s // NB + 1 < NGRP:
          idx_copy(s // NB + 1).start()
        out_copy(s, b).start()
      for b in range(NBUF):
        s = s0 + b
        out_copy(s, b).wait()
        if s + NBUF < NSLAB:
          in_copy(s + NBUF, b).start()

  return k(ev, iv)


def kernel(embed, indexes, mask_num):
  del mask_num
  ev = jnp.transpose(embed, (1, 2, 0)).reshape(EMBED, BATCH)
  iv = jnp.transpose(indexes.astype(jnp.int32), (1, 0))
  ov = _sc_mask_copy(ev, iv)
  return jnp.transpose(ov.reshape(FIELDS, EDIM, BATCH), (2, 0, 1))
